# baseline (speedup 1.0000x reference)
import functools

import jax
import jax.numpy as jnp
from jax import lax
from jax.experimental import pallas as pl
from jax.experimental.pallas import tpu as pltpu
from jax.experimental.pallas import tpu_sc as plsc

_NUM_CORES = 2
_NUM_SUBCORES = 16
_NW = _NUM_CORES * _NUM_SUBCORES
_ILIM = 128
_CHUNK = 256
_NBUF = 2
_LOOK = 1


@jax.jit
def _embed_flat(idx, table):
    total = idx.shape[0]
    _, d = table.shape
    b_per_w = total // _NW
    assert total % _NW == 0 and b_per_w % _CHUNK == 0
    nchunk = b_per_w // _CHUNK
    assert (nchunk - _NBUF) % _NBUF == 0 and nchunk >= 2 * _NBUF

    mesh = plsc.VectorSubcoreMesh(core_axis_name="c", subcore_axis_name="s")

    @functools.partial(
        pl.kernel,
        mesh=mesh,
        out_type=jax.ShapeDtypeStruct((total, d), jnp.float32),
        scratch_types=[
            pltpu.VMEM((b_per_w,), jnp.int32),
            pltpu.VMEM((_NBUF, _CHUNK, d), jnp.float32),
            pltpu.SemaphoreType.DMA((_NBUF,)),
            pltpu.SemaphoreType.DMA((_NBUF,)),
        ],
    )
    def k(idx_hbm, table_hbm, out_hbm, idx_v, rows_v, sem_g, sem_w):
        wid = lax.axis_index("s") * _NUM_CORES + lax.axis_index("c")
        base = wid * b_per_w
        pltpu.sync_copy(idx_hbm.at[pl.ds(base, b_per_w)], idx_v)

        def g_start(g, b):
            for o in range(0, _CHUNK, _ILIM):
                pltpu.async_copy(
                    table_hbm.at[idx_v.at[pl.ds(g * _CHUNK + o, _ILIM)]],
                    rows_v.at[b, pl.ds(o, _ILIM)],
                    sem_g.at[b],
                )

        def g_wait(g, b):
            for o in range(0, _CHUNK, _ILIM):
                pltpu.make_async_copy(
                    table_hbm.at[idx_v.at[pl.ds(g * _CHUNK + o, _ILIM)]],
                    rows_v.at[b, pl.ds(o, _ILIM)],
                    sem_g.at[b],
                ).wait()

        def w_start(g, b):
            pltpu.async_copy(
                rows_v.at[b],
                out_hbm.at[pl.ds(base + g * _CHUNK, _CHUNK)],
                sem_w.at[b],
            )

        def w_wait(b):
            pltpu.make_async_copy(
                rows_v.at[b],
                out_hbm.at[pl.ds(base, _CHUNK)],
                sem_w.at[b],
            ).wait()

        for g in range(_LOOK):
            g_start(g, g % _NBUF)
        for g in range(_NBUF - _LOOK):
            g_start(g + _LOOK, (g + _LOOK) % _NBUF)
            g_wait(g, g % _NBUF)
            w_start(g, g % _NBUF)

        g0 = _NBUF - _LOOK

        def body(t, carry):
            go = g0 + t * _NBUF
            for i in range(_NBUF):
                g = go + i
                b = (g0 + i) % _NBUF
                bn = (b + _LOOK) % _NBUF
                w_wait(bn)
                g_start(g + _LOOK, bn)
                g_wait(g, b)
                w_start(g, b)
            return carry

        lax.fori_loop(0, (nchunk - _NBUF) // _NBUF, body, 0)

        for g in range(nchunk - _LOOK, nchunk):
            b = g % _NBUF
            g_wait(g, b)
            w_start(g, b)
        for b in range(_NBUF):
            w_wait(b)

    return k(idx, table)


def kernel(x, table):
    b, h = x.shape
    out = _embed_flat(x.reshape(b * h), table)
    return out.reshape(b, h, table.shape[1])


# speedup vs baseline: 1.0040x; 1.0040x over previous
import functools

import jax
import jax.numpy as jnp
from jax import lax
from jax.experimental import pallas as pl
from jax.experimental.pallas import tpu as pltpu
from jax.experimental.pallas import tpu_sc as plsc

_NUM_CORES = 2
_NUM_SUBCORES = 16
_NW = _NUM_CORES * _NUM_SUBCORES
_ILIM = 128
_CHUNK = 128
_NBUF = 5
_LOOK = 2


@jax.jit
def _embed_flat(idx, table):
    total = idx.shape[0]
    _, d = table.shape
    b_per_w = total // _NW
    assert total % _NW == 0 and b_per_w % _CHUNK == 0
    nchunk = b_per_w // _CHUNK
    assert (nchunk - _NBUF) % _NBUF == 0 and nchunk >= 2 * _NBUF

    mesh = plsc.VectorSubcoreMesh(core_axis_name="c", subcore_axis_name="s")

    @functools.partial(
        pl.kernel,
        mesh=mesh,
        out_type=jax.ShapeDtypeStruct((total, d), jnp.float32),
        scratch_types=[
            pltpu.VMEM((b_per_w,), jnp.int32),
            pltpu.VMEM((_NBUF, _CHUNK, d), jnp.float32),
            pltpu.SemaphoreType.DMA((_NBUF,)),
            pltpu.SemaphoreType.DMA((_NBUF,)),
        ],
    )
    def k(idx_hbm, table_hbm, out_hbm, idx_v, rows_v, sem_g, sem_w):
        wid = lax.axis_index("s") * _NUM_CORES + lax.axis_index("c")
        base = wid * b_per_w
        pltpu.sync_copy(idx_hbm.at[pl.ds(base, b_per_w)], idx_v)

        def g_start(g, b):
            for o in range(0, _CHUNK, _ILIM):
                pltpu.async_copy(
                    table_hbm.at[idx_v.at[pl.ds(g * _CHUNK + o, _ILIM)]],
                    rows_v.at[b, pl.ds(o, _ILIM)],
                    sem_g.at[b],
                )

        def g_wait(g, b):
            for o in range(0, _CHUNK, _ILIM):
                pltpu.make_async_copy(
                    table_hbm.at[idx_v.at[pl.ds(g * _CHUNK + o, _ILIM)]],
                    rows_v.at[b, pl.ds(o, _ILIM)],
                    sem_g.at[b],
                ).wait()

        def w_start(g, b):
            pltpu.async_copy(
                rows_v.at[b],
                out_hbm.at[pl.ds(base + g * _CHUNK, _CHUNK)],
                sem_w.at[b],
            )

        def w_wait(b):
            pltpu.make_async_copy(
                rows_v.at[b],
                out_hbm.at[pl.ds(base, _CHUNK)],
                sem_w.at[b],
            ).wait()

        for g in range(_LOOK):
            g_start(g, g % _NBUF)
        for g in range(_NBUF - _LOOK):
            g_start(g + _LOOK, (g + _LOOK) % _NBUF)
            g_wait(g, g % _NBUF)
            w_start(g, g % _NBUF)

        g0 = _NBUF - _LOOK

        def body(t, carry):
            go = g0 + t * _NBUF
            for i in range(_NBUF):
                g = go + i
                b = (g0 + i) % _NBUF
                bn = (b + _LOOK) % _NBUF
                w_wait(bn)
                g_start(g + _LOOK, bn)
                g_wait(g, b)
                w_start(g, b)
            return carry

        lax.fori_loop(0, (nchunk - _NBUF) // _NBUF, body, 0)

        for g in range(nchunk - _LOOK, nchunk):
            b = g % _NBUF
            g_wait(g, b)
            w_start(g, b)
        for b in range(_NBUF):
            w_wait(b)

    return k(idx, table)


def kernel(x, table):
    b, h = x.shape
    out = _embed_flat(x.reshape(b * h), table)
    return out.reshape(b, h, table.shape[1])
